# baseline (speedup 1.0000x reference)
import functools

import jax
import jax.numpy as jnp
from jax import lax
from jax.experimental import pallas as pl
from jax.experimental.pallas import tpu as pltpu
from jax.experimental.pallas import tpu_sc as plsc

_NB = 4096
_CONFS = 32
_R2, _R3, _R4 = 256, 512, 768

_SC_CORES = 2
_SC_SUBCORES = 16
_SC_TILES = _SC_CORES * _SC_SUBCORES
_SEG_PER_TILE = _NB // _SC_TILES
_JJ_LEN = _NB + 8


def _cumsum_rows(x):
    rows = x.shape[0]
    k = 1
    while k < rows:
        shifted = jnp.concatenate(
            [jnp.zeros((k, x.shape[1]), x.dtype), x[: rows - k]], axis=0)
        x = x + shifted
        k *= 2
    return x


def _prefix_body(k2_ref, eq2_ref, x2_ref, k3_ref, eq3_ref, x3_ref,
                 k4_ref, x4_ref, p2_ref, p3_ref, p4_ref,
                 c2_ref, c3_ref, c4_ref):
    b = pl.program_id(0)

    @pl.when(b == 0)
    def _zero():
        c2_ref[...] = jnp.zeros_like(c2_ref)
        c3_ref[...] = jnp.zeros_like(c3_ref)
        c4_ref[...] = jnp.zeros_like(c4_ref)
        p2_ref[...] = jnp.zeros_like(p2_ref)
        p3_ref[...] = jnp.zeros_like(p3_ref)
        p4_ref[...] = jnp.zeros_like(p4_ref)

    @pl.when(b > 0)
    def _compute():
        def emit(p_ref, c_ref, s, rows):
            p_ref[...] = jnp.concatenate(
                [s + c_ref[...],
                 jnp.zeros((rows, 128 - _CONFS), jnp.float32)], axis=1)
            c_ref[...] = c_ref[...] + s[rows - 1:rows, :]

        d2 = x2_ref[...] - eq2_ref[...]
        emit(p2_ref, c2_ref, _cumsum_rows(0.5 * k2_ref[...] * (d2 * d2)), _R2)

        d3 = x3_ref[...] - eq3_ref[...]
        emit(p3_ref, c3_ref, _cumsum_rows(0.5 * k3_ref[...] * (d3 * d3)), _R3)

        c1 = jnp.cos(x4_ref[...])
        t2 = 2.0 * c1 * c1 - 1.0
        t3 = 2.0 * c1 * t2 - c1
        t4 = 2.0 * c1 * t3 - t2
        t5 = 2.0 * c1 * t4 - t3
        t6 = 2.0 * c1 * t5 - t4
        k4 = k4_ref[...]
        e4 = (k4[:, 0:1] * c1 + k4[:, 1:2] * t2 + k4[:, 2:3] * t3
              + k4[:, 3:4] * t4 + k4[:, 4:5] * t5 + k4[:, 5:6] * t6)
        emit(p4_ref, c4_ref, _cumsum_rows(e4), _R4)


def _prefix(k2, eq2, x2, k3, eq3, x3, k4, x4):
    n2, n3, n4 = x2.shape[0], x3.shape[0], x4.shape[0]
    steps = n2 // _R2 + 1

    def shifted(r):
        return lambda b: (jnp.maximum(b - 1, 0), 0)

    return pl.pallas_call(
        _prefix_body,
        grid=(steps,),
        in_specs=[
            pl.BlockSpec((_R2, 1), shifted(_R2)),
            pl.BlockSpec((_R2, 1), shifted(_R2)),
            pl.BlockSpec((_R2, _CONFS), shifted(_R2)),
            pl.BlockSpec((_R3, 1), shifted(_R3)),
            pl.BlockSpec((_R3, 1), shifted(_R3)),
            pl.BlockSpec((_R3, _CONFS), shifted(_R3)),
            pl.BlockSpec((_R4, 6), shifted(_R4)),
            pl.BlockSpec((_R4, _CONFS), shifted(_R4)),
        ],
        out_specs=[
            pl.BlockSpec((_R2, 128), lambda b: (b, 0)),
            pl.BlockSpec((_R3, 128), lambda b: (b, 0)),
            pl.BlockSpec((_R4, 128), lambda b: (b, 0)),
        ],
        out_shape=[
            jax.ShapeDtypeStruct((n2 + _R2, 128), jnp.float32),
            jax.ShapeDtypeStruct((n3 + _R3, 128), jnp.float32),
            jax.ShapeDtypeStruct((n4 + _R4, 128), jnp.float32),
        ],
        scratch_shapes=[
            pltpu.VMEM((1, _CONFS), jnp.float32),
            pltpu.VMEM((1, _CONFS), jnp.float32),
            pltpu.VMEM((1, _CONFS), jnp.float32),
        ],
    )(k2, eq2, x2, k3, eq3, x3, k4, x4)


def _sc_body(p2_hbm, p3_hbm, p4_hbm, jj2_hbm, jj3_hbm, jj4_hbm, out_hbm,
             jj2_v, jj3_v, jj4_v, g2_v, g3_v, g4_v, out_v, sem):
    wid = lax.axis_index("s") * _SC_CORES + lax.axis_index("c")
    base = wid * _SEG_PER_TILE

    pltpu.sync_copy(jj2_hbm.at[pl.ds(base, 136)], jj2_v)
    pltpu.sync_copy(jj3_hbm.at[pl.ds(base, 136)], jj3_v)
    pltpu.sync_copy(jj4_hbm.at[pl.ds(base, 136)], jj4_v)

    copies = []
    for p_hbm, jj_v, g_v in ((p2_hbm, jj2_v, g2_v),
                             (p3_hbm, jj3_v, g3_v),
                             (p4_hbm, jj4_v, g4_v)):
        copies.append(pltpu.async_copy(
            p_hbm.at[jj_v.at[pl.ds(0, 128)]], g_v.at[pl.ds(0, 128)], sem))
        copies.append(pltpu.async_copy(
            p_hbm.at[jj_v.at[pl.ds(128, 8)]], g_v.at[pl.ds(128, 8)], sem))
    for cp in copies:
        cp.wait()

    def row(i, carry):
        for cb in (0, 16):
            v = (g2_v[i + 1, pl.ds(cb, 16)] - g2_v[i, pl.ds(cb, 16)]
                 + g3_v[i + 1, pl.ds(cb, 16)] - g3_v[i, pl.ds(cb, 16)]
                 + g4_v[i + 1, pl.ds(cb, 16)] - g4_v[i, pl.ds(cb, 16)])
            out_v[i, pl.ds(cb, 16)] = v
        return carry

    lax.fori_loop(0, _SEG_PER_TILE, row, 0)
    pltpu.sync_copy(out_v, out_hbm.at[pl.ds(base, _SEG_PER_TILE)])


def _sc_gather(p2, p3, p4, jj2, jj3, jj4):
    mesh = plsc.VectorSubcoreMesh(
        core_axis_name="c", subcore_axis_name="s",
        num_cores=_SC_CORES, num_subcores=_SC_SUBCORES)
    fn = pl.kernel(
        _sc_body,
        out_type=jax.ShapeDtypeStruct((_NB, _CONFS), jnp.float32),
        mesh=mesh,
        scratch_types=[
            pltpu.VMEM((136,), jnp.int32),
            pltpu.VMEM((136,), jnp.int32),
            pltpu.VMEM((136,), jnp.int32),
            pltpu.VMEM((136, 128), jnp.float32),
            pltpu.VMEM((136, 128), jnp.float32),
            pltpu.VMEM((136, 128), jnp.float32),
            pltpu.VMEM((_SEG_PER_TILE, _CONFS), jnp.float32),
            pltpu.SemaphoreType.DMA,
        ],
    )
    return fn(p2, p3, p4, jj2, jj3, jj4)


def _boundaries(seg, n, r):
    s = jnp.arange(_NB, dtype=jnp.int32)
    j = jnp.searchsorted(seg.astype(jnp.int32), s, side="left")
    j = jnp.concatenate([j, jnp.full((_JJ_LEN - _NB,), n, dtype=j.dtype)])
    return (j + (r - 1)).astype(jnp.int32)


def kernel(k2, eq2, x2, seg2, k3, eq3, x3, seg3, k4, x4, seg4):
    n2, n3, n4 = x2.shape[0], x3.shape[0], x4.shape[0]
    p2, p3, p4 = _prefix(
        k2.reshape(n2, 1), eq2.reshape(n2, 1), x2,
        k3.reshape(n3, 1), eq3.reshape(n3, 1), x3,
        k4, x4)
    jj2 = _boundaries(seg2, n2, _R2)
    jj3 = _boundaries(seg3, n3, _R3)
    jj4 = _boundaries(seg4, n4, _R4)
    return _sc_gather(p2, p3, p4, jj2, jj3, jj4)


# speedup vs baseline: 1.2821x; 1.2821x over previous
import jax
import jax.numpy as jnp
from jax import lax
from jax.experimental import pallas as pl
from jax.experimental.pallas import tpu as pltpu
from jax.experimental.pallas import tpu_sc as plsc

_NB = 4096
_CONFS = 32
_R2, _R3, _R4 = 256, 512, 768

_SC_CORES = 2
_SC_SUBCORES = 16
_SC_TILES = _SC_CORES * _SC_SUBCORES
_SEG_PER_TILE = _NB // _SC_TILES
_NQ = 144
_NCHUNK = _NQ // 16


def _cumsum_rows(x):
    rows = x.shape[0]
    k = 1
    while k < rows:
        shifted = jnp.concatenate(
            [jnp.zeros((k, x.shape[1]), x.dtype), x[: rows - k]], axis=0)
        x = x + shifted
        k *= 2
    return x


def _prefix_body(k2_ref, eq2_ref, x2_ref, k3_ref, eq3_ref, x3_ref,
                 k4_ref, x4_ref, p2_ref, p3_ref, p4_ref,
                 c2_ref, c3_ref, c4_ref):
    b = pl.program_id(0)

    @pl.when(b == 0)
    def _zero():
        c2_ref[...] = jnp.zeros_like(c2_ref)
        c3_ref[...] = jnp.zeros_like(c3_ref)
        c4_ref[...] = jnp.zeros_like(c4_ref)
        p2_ref[...] = jnp.zeros_like(p2_ref)
        p3_ref[...] = jnp.zeros_like(p3_ref)
        p4_ref[...] = jnp.zeros_like(p4_ref)

    @pl.when(b > 0)
    def _compute():
        def emit(p_ref, c_ref, s, rows):
            p_ref[...] = jnp.concatenate(
                [s + c_ref[...],
                 jnp.zeros((rows, 128 - _CONFS), jnp.float32)], axis=1)
            c_ref[...] = c_ref[...] + s[rows - 1:rows, :]

        d2 = x2_ref[...] - eq2_ref[...]
        emit(p2_ref, c2_ref, _cumsum_rows(0.5 * k2_ref[...] * (d2 * d2)), _R2)

        d3 = x3_ref[...] - eq3_ref[...]
        emit(p3_ref, c3_ref, _cumsum_rows(0.5 * k3_ref[...] * (d3 * d3)), _R3)

        c1 = jnp.cos(x4_ref[...])
        t2 = 2.0 * c1 * c1 - 1.0
        t3 = 2.0 * c1 * t2 - c1
        t4 = 2.0 * c1 * t3 - t2
        t5 = 2.0 * c1 * t4 - t3
        t6 = 2.0 * c1 * t5 - t4
        k4 = k4_ref[...]
        e4 = (k4[:, 0:1] * c1 + k4[:, 1:2] * t2 + k4[:, 2:3] * t3
              + k4[:, 3:4] * t4 + k4[:, 4:5] * t5 + k4[:, 5:6] * t6)
        emit(p4_ref, c4_ref, _cumsum_rows(e4), _R4)


def _prefix(k2, eq2, x2, k3, eq3, x3, k4, x4):
    n2, n3, n4 = x2.shape[0], x3.shape[0], x4.shape[0]
    steps = n2 // _R2 + 1

    def shifted(r):
        return lambda b: (jnp.maximum(b - 1, 0), 0)

    return pl.pallas_call(
        _prefix_body,
        grid=(steps,),
        in_specs=[
            pl.BlockSpec((_R2, 1), shifted(_R2)),
            pl.BlockSpec((_R2, 1), shifted(_R2)),
            pl.BlockSpec((_R2, _CONFS), shifted(_R2)),
            pl.BlockSpec((_R3, 1), shifted(_R3)),
            pl.BlockSpec((_R3, 1), shifted(_R3)),
            pl.BlockSpec((_R3, _CONFS), shifted(_R3)),
            pl.BlockSpec((_R4, 6), shifted(_R4)),
            pl.BlockSpec((_R4, _CONFS), shifted(_R4)),
        ],
        out_specs=[
            pl.BlockSpec((_R2, 128), lambda b: (b, 0)),
            pl.BlockSpec((_R3, 128), lambda b: (b, 0)),
            pl.BlockSpec((_R4, 128), lambda b: (b, 0)),
        ],
        out_shape=[
            jax.ShapeDtypeStruct((n2 + _R2, 128), jnp.float32),
            jax.ShapeDtypeStruct((n3 + _R3, 128), jnp.float32),
            jax.ShapeDtypeStruct((n4 + _R4, 128), jnp.float32),
        ],
        scratch_shapes=[
            pltpu.VMEM((1, _CONFS), jnp.float32),
            pltpu.VMEM((1, _CONFS), jnp.float32),
            pltpu.VMEM((1, _CONFS), jnp.float32),
        ],
    )(k2, eq2, x2, k3, eq3, x3, k4, x4)


def _sc_body(p2_hbm, p3_hbm, p4_hbm, s2d2_hbm, s2d3_hbm, s2d4_hbm,
             sub2_hbm, sub3_hbm, sub4_hbm, out_hbm,
             sub2_v, sub3_v, sub4_v, probe_v, segrows_v,
             jj2_v, jj3_v, jj4_v, g2_v, g3_v, g4_v, out_v,
             sem_s, sem_p):
    wid = lax.axis_index("s") * _SC_CORES + lax.axis_index("c")
    base = wid * _SEG_PER_TILE

    pltpu.sync_copy(sub2_hbm, sub2_v)
    pltpu.sync_copy(sub3_hbm, sub3_v)
    pltpu.sync_copy(sub4_hbm, sub4_v)

    p_copies = []
    for sub_v, nsub, s2d_hbm, jj_v, p_hbm, g_v, radd in (
            (sub2_v, 1024, s2d2_hbm, jj2_v, p2_hbm, g2_v, _R2 - 1),
            (sub3_v, 2048, s2d3_hbm, jj3_v, p3_hbm, g3_v, _R3 - 1),
            (sub4_v, 3072, s2d4_hbm, jj4_v, p4_hbm, g4_v, _R4 - 1)):

        def coarse(c, carry, sub_v=sub_v, nsub=nsub):
            s = base + c * 16 + lax.iota(jnp.int32, 16)
            lo = jnp.zeros((16,), jnp.int32)
            hi = jnp.full((16,), nsub, jnp.int32)
            for _ in range(12):
                mid = jnp.minimum((lo + hi) >> 1, nsub - 1)
                v = plsc.load_gather(sub_v, [mid])
                below = v < s
                lo = jnp.where(below, mid + 1, lo)
                hi = jnp.where(below, hi, mid)
            probe_v[pl.ds(c * 16, 16)] = jnp.maximum(lo - 1, 0)
            return carry

        lax.fori_loop(0, _NCHUNK, coarse, 0)

        pltpu.async_copy(s2d_hbm.at[probe_v.at[pl.ds(0, 128)]],
                         segrows_v.at[pl.ds(0, 128)], sem_s).wait()
        pltpu.async_copy(s2d_hbm.at[probe_v.at[pl.ds(128, 16)]],
                         segrows_v.at[pl.ds(128, 16)], sem_s).wait()

        def fine(c, carry, jj_v=jj_v, radd=radd):
            s = base + c * 16 + lax.iota(jnp.int32, 16)
            row = c * 16 + lax.iota(jnp.int32, 16)
            qb = probe_v[pl.ds(c * 16, 16)]
            lo = jnp.zeros((16,), jnp.int32)
            hi = jnp.full((16,), 128, jnp.int32)
            for _ in range(8):
                mid = jnp.minimum((lo + hi) >> 1, 127)
                v = plsc.load_gather(segrows_v, [row, mid])
                below = v < s
                lo = jnp.where(below, mid + 1, lo)
                hi = jnp.where(below, hi, mid)
            jj_v[pl.ds(c * 16, 16)] = qb * 128 + lo + radd
            return carry

        lax.fori_loop(0, _NCHUNK, fine, 0)

        p_copies.append(pltpu.async_copy(
            p_hbm.at[jj_v.at[pl.ds(0, 128)]], g_v.at[pl.ds(0, 128)], sem_p))
        p_copies.append(pltpu.async_copy(
            p_hbm.at[jj_v.at[pl.ds(128, 16)]], g_v.at[pl.ds(128, 16)], sem_p))

    for cp in p_copies:
        cp.wait()

    def row(i, carry):
        for cb in (0, 16):
            v = (g2_v[i + 1, pl.ds(cb, 16)] - g2_v[i, pl.ds(cb, 16)]
                 + g3_v[i + 1, pl.ds(cb, 16)] - g3_v[i, pl.ds(cb, 16)]
                 + g4_v[i + 1, pl.ds(cb, 16)] - g4_v[i, pl.ds(cb, 16)])
            out_v[i, pl.ds(cb, 16)] = v
        return carry

    lax.fori_loop(0, _SEG_PER_TILE, row, 0)
    pltpu.sync_copy(out_v, out_hbm.at[pl.ds(base, _SEG_PER_TILE)])


def _sc_gather(p2, p3, p4, s2d2, s2d3, s2d4, sub2, sub3, sub4):
    mesh = plsc.VectorSubcoreMesh(
        core_axis_name="c", subcore_axis_name="s",
        num_cores=_SC_CORES, num_subcores=_SC_SUBCORES)
    fn = pl.kernel(
        _sc_body,
        out_type=jax.ShapeDtypeStruct((_NB, _CONFS), jnp.float32),
        mesh=mesh,
        compiler_params=pltpu.CompilerParams(needs_layout_passes=False),
        scratch_types=[
            pltpu.VMEM((1024,), jnp.int32),
            pltpu.VMEM((2048,), jnp.int32),
            pltpu.VMEM((3072,), jnp.int32),
            pltpu.VMEM((_NQ,), jnp.int32),
            pltpu.VMEM((_NQ, 128), jnp.int32),
            pltpu.VMEM((_NQ,), jnp.int32),
            pltpu.VMEM((_NQ,), jnp.int32),
            pltpu.VMEM((_NQ,), jnp.int32),
            pltpu.VMEM((_NQ, 128), jnp.float32),
            pltpu.VMEM((_NQ, 128), jnp.float32),
            pltpu.VMEM((_NQ, 128), jnp.float32),
            pltpu.VMEM((_SEG_PER_TILE, _CONFS), jnp.float32),
            pltpu.SemaphoreType.DMA,
            pltpu.SemaphoreType.DMA,
        ],
    )
    return fn(p2, p3, p4, s2d2, s2d3, s2d4, sub2, sub3, sub4)


def kernel(k2, eq2, x2, seg2, k3, eq3, x3, seg3, k4, x4, seg4):
    n2, n3, n4 = x2.shape[0], x3.shape[0], x4.shape[0]
    p2, p3, p4 = _prefix(
        k2.reshape(n2, 1), eq2.reshape(n2, 1), x2,
        k3.reshape(n3, 1), eq3.reshape(n3, 1), x3,
        k4, x4)
    seg2 = seg2.astype(jnp.int32)
    seg3 = seg3.astype(jnp.int32)
    seg4 = seg4.astype(jnp.int32)
    return _sc_gather(
        p2, p3, p4,
        seg2.reshape(n2 // 128, 128),
        seg3.reshape(n3 // 128, 128),
        seg4.reshape(n4 // 128, 128),
        seg2[::128], seg3[::128], seg4[::128])


# speedup vs baseline: 1.7202x; 1.3418x over previous
import jax
import jax.numpy as jnp
from jax import lax
from jax.experimental import pallas as pl
from jax.experimental.pallas import tpu as pltpu
from jax.experimental.pallas import tpu_sc as plsc

_NB = 4096
_CONFS = 32
_R2, _R3, _R4 = 512, 1024, 1536

_SC_CORES = 2
_SC_SUBCORES = 16
_SC_TILES = _SC_CORES * _SC_SUBCORES
_SEG_PER_TILE = _NB // _SC_TILES
_NQ = 144
_NCHUNK = _NQ // 16


def _cumsum_rows(x):
    rows = x.shape[0]
    k = 1
    while k < rows:
        shifted = jnp.concatenate(
            [jnp.zeros((k, x.shape[1]), x.dtype), x[: rows - k]], axis=0)
        x = x + shifted
        k *= 2
    return x


def _shift_lanes(x, k):
    return jnp.concatenate(
        [jnp.zeros((x.shape[0], k), x.dtype), x[:, : x.shape[1] - k]], axis=1)


def _prefix_body(k2_ref, eq2_ref, x2_ref, k3_ref, eq3_ref, x3_ref,
                 k4_ref, x4_ref, p2_ref, p3_ref, p4_ref,
                 c2_ref, c3_ref, c4_ref):
    b = pl.program_id(0)

    @pl.when(b == 0)
    def _zero():
        c2_ref[...] = jnp.zeros_like(c2_ref)
        c3_ref[...] = jnp.zeros_like(c3_ref)
        c4_ref[...] = jnp.zeros_like(c4_ref)
        p2_ref[...] = jnp.zeros_like(p2_ref)
        p3_ref[...] = jnp.zeros_like(p3_ref)
        p4_ref[...] = jnp.zeros_like(p4_ref)

    @pl.when(b > 0)
    def _compute():
        def pack(x, rows):
            q = rows // 4
            return jnp.concatenate(
                [x[0:q], x[q:2 * q], x[2 * q:3 * q], x[3 * q:4 * q]], axis=1)

        def pack_col(col, rows):
            q = rows // 4
            return jnp.concatenate(
                [jnp.broadcast_to(col[k * q:(k + 1) * q], (q, _CONFS))
                 for k in range(4)], axis=1)

        def emit(p_ref, c_ref, y, q):
            y = _cumsum_rows(y)
            last = y[q - 1:q, :]
            a = _shift_lanes(last, 32)
            off = a + _shift_lanes(a, 32) + _shift_lanes(a, 64)
            p_ref[...] = y + off + c_ref[...]
            tot = (off + last)[:, 96:128]
            c_ref[...] = c_ref[...] + jnp.concatenate(
                [tot, tot, tot, tot], axis=1)

        d2 = pack(x2_ref[...], _R2) - pack_col(eq2_ref[...], _R2)
        emit(p2_ref, c2_ref,
             0.5 * pack_col(k2_ref[...], _R2) * (d2 * d2), _R2 // 4)

        d3 = pack(x3_ref[...], _R3) - pack_col(eq3_ref[...], _R3)
        emit(p3_ref, c3_ref,
             0.5 * pack_col(k3_ref[...], _R3) * (d3 * d3), _R3 // 4)

        c1 = jnp.cos(pack(x4_ref[...], _R4))
        t2 = 2.0 * c1 * c1 - 1.0
        t3 = 2.0 * c1 * t2 - c1
        t4 = 2.0 * c1 * t3 - t2
        t5 = 2.0 * c1 * t4 - t3
        t6 = 2.0 * c1 * t5 - t4
        k4 = k4_ref[...]
        e4 = (pack_col(k4[:, 0:1], _R4) * c1
              + pack_col(k4[:, 1:2], _R4) * t2
              + pack_col(k4[:, 2:3], _R4) * t3
              + pack_col(k4[:, 3:4], _R4) * t4
              + pack_col(k4[:, 4:5], _R4) * t5
              + pack_col(k4[:, 5:6], _R4) * t6)
        emit(p4_ref, c4_ref, e4, _R4 // 4)


def _prefix(k2, eq2, x2, k3, eq3, x3, k4, x4):
    n2, n3, n4 = x2.shape[0], x3.shape[0], x4.shape[0]
    steps = n2 // _R2 + 1

    def shifted(r):
        return lambda b: (jnp.maximum(b - 1, 0), 0)

    return pl.pallas_call(
        _prefix_body,
        grid=(steps,),
        in_specs=[
            pl.BlockSpec((_R2, 1), shifted(_R2)),
            pl.BlockSpec((_R2, 1), shifted(_R2)),
            pl.BlockSpec((_R2, _CONFS), shifted(_R2)),
            pl.BlockSpec((_R3, 1), shifted(_R3)),
            pl.BlockSpec((_R3, 1), shifted(_R3)),
            pl.BlockSpec((_R3, _CONFS), shifted(_R3)),
            pl.BlockSpec((_R4, 6), shifted(_R4)),
            pl.BlockSpec((_R4, _CONFS), shifted(_R4)),
        ],
        out_specs=[
            pl.BlockSpec((_R2 // 4, 128), lambda b: (b, 0)),
            pl.BlockSpec((_R3 // 4, 128), lambda b: (b, 0)),
            pl.BlockSpec((_R4 // 4, 128), lambda b: (b, 0)),
        ],
        out_shape=[
            jax.ShapeDtypeStruct(((n2 + _R2) // 4, 128), jnp.float32),
            jax.ShapeDtypeStruct(((n3 + _R3) // 4, 128), jnp.float32),
            jax.ShapeDtypeStruct(((n4 + _R4) // 4, 128), jnp.float32),
        ],
        scratch_shapes=[
            pltpu.VMEM((1, 128), jnp.float32),
            pltpu.VMEM((1, 128), jnp.float32),
            pltpu.VMEM((1, 128), jnp.float32),
        ],
    )(k2, eq2, x2, k3, eq3, x3, k4, x4)


def _map2(jj):
    return (jj >> 9) * 128 + (jj & 127), ((jj & 511) >> 7) * 32


def _map3(jj):
    return (jj >> 10) * 256 + (jj & 255), ((jj & 1023) >> 8) * 32


def _map4(jj):
    b = ((jj >> 9) * 21846) >> 16
    ul = jj - b * 1536
    k = ((ul >= 384).astype(jnp.int32) + (ul >= 768).astype(jnp.int32)
         + (ul >= 1152).astype(jnp.int32))
    return b * 384 + (ul - k * 384), k * 32


def _sc_body(p2_hbm, p3_hbm, p4_hbm, s2d2_hbm, s2d3_hbm, s2d4_hbm,
             sub2_hbm, sub3_hbm, sub4_hbm, out_hbm,
             sub2_v, sub3_v, sub4_v, probe_v, segrows_v,
             km2_v, km3_v, km4_v, qr2_v, qr3_v, qr4_v,
             g2_v, g3_v, g4_v, out_v, sem_s, sem_p):
    wid = lax.axis_index("s") * _SC_CORES + lax.axis_index("c")
    base = wid * _SEG_PER_TILE

    pltpu.sync_copy(sub2_hbm, sub2_v)
    pltpu.sync_copy(sub3_hbm, sub3_v)
    pltpu.sync_copy(sub4_hbm, sub4_v)

    p_copies = []
    for sub_v, nsub, s2d_hbm, km_v, qr_v, p_hbm, g_v, radd, pmap in (
            (sub2_v, 1024, s2d2_hbm, km2_v, qr2_v, p2_hbm, g2_v, _R2 - 1, _map2),
            (sub3_v, 2048, s2d3_hbm, km3_v, qr3_v, p3_hbm, g3_v, _R3 - 1, _map3),
            (sub4_v, 3072, s2d4_hbm, km4_v, qr4_v, p4_hbm, g4_v, _R4 - 1, _map4)):

        def coarse(c, carry, sub_v=sub_v, nsub=nsub):
            s = base + c * 16 + lax.iota(jnp.int32, 16)
            lo = jnp.zeros((16,), jnp.int32)
            hi = jnp.full((16,), nsub, jnp.int32)
            for _ in range(12):
                mid = jnp.minimum((lo + hi) >> 1, nsub - 1)
                v = plsc.load_gather(sub_v, [mid])
                below = v < s
                lo = jnp.where(below, mid + 1, lo)
                hi = jnp.where(below, hi, mid)
            probe_v[pl.ds(c * 16, 16)] = jnp.maximum(lo - 1, 0)
            return carry

        lax.fori_loop(0, _NCHUNK, coarse, 0)

        pltpu.async_copy(s2d_hbm.at[probe_v.at[pl.ds(0, 128)]],
                         segrows_v.at[pl.ds(0, 128)], sem_s).wait()
        pltpu.async_copy(s2d_hbm.at[probe_v.at[pl.ds(128, 16)]],
                         segrows_v.at[pl.ds(128, 16)], sem_s).wait()

        def fine(c, carry, km_v=km_v, qr_v=qr_v, radd=radd, pmap=pmap):
            s = base + c * 16 + lax.iota(jnp.int32, 16)
            row = c * 16 + lax.iota(jnp.int32, 16)
            qb = probe_v[pl.ds(c * 16, 16)]
            lo = jnp.zeros((16,), jnp.int32)
            hi = jnp.full((16,), 128, jnp.int32)
            for _ in range(8):
                mid = jnp.minimum((lo + hi) >> 1, 127)
                v = plsc.load_gather(segrows_v, [row, mid])
                below = v < s
                lo = jnp.where(below, mid + 1, lo)
                hi = jnp.where(below, hi, mid)
            jj = qb * 128 + lo + radd
            prow, km = pmap(jj)
            qr_v[pl.ds(c * 16, 16)] = prow
            km_v[pl.ds(c * 16, 16)] = km
            return carry

        lax.fori_loop(0, _NCHUNK, fine, 0)

        p_copies.append(pltpu.async_copy(
            p_hbm.at[qr_v.at[pl.ds(0, 128)]], g_v.at[pl.ds(0, 128)], sem_p))
        p_copies.append(pltpu.async_copy(
            p_hbm.at[qr_v.at[pl.ds(128, 16)]], g_v.at[pl.ds(128, 16)], sem_p))

    for cp in p_copies:
        cp.wait()

    def diff(c, carry):
        rlo = c * 16 + lax.iota(jnp.int32, 16)
        rhi = rlo + 1
        jmlo2 = km2_v[pl.ds(c * 16, 16)]
        jmhi2 = km2_v[pl.ds(c * 16 + 1, 16)]
        jmlo3 = km3_v[pl.ds(c * 16, 16)]
        jmhi3 = km3_v[pl.ds(c * 16 + 1, 16)]
        jmlo4 = km4_v[pl.ds(c * 16, 16)]
        jmhi4 = km4_v[pl.ds(c * 16 + 1, 16)]
        for cc in range(_CONFS):
            col = jnp.full((16,), cc, jnp.int32)
            acc = (plsc.load_gather(g2_v, [rhi, jmhi2 + cc])
                   - plsc.load_gather(g2_v, [rlo, jmlo2 + cc])
                   + plsc.load_gather(g3_v, [rhi, jmhi3 + cc])
                   - plsc.load_gather(g3_v, [rlo, jmlo3 + cc])
                   + plsc.load_gather(g4_v, [rhi, jmhi4 + cc])
                   - plsc.load_gather(g4_v, [rlo, jmlo4 + cc]))
            plsc.store_scatter(out_v, [rlo, col], acc)
        return carry

    lax.fori_loop(0, _SEG_PER_TILE // 16, diff, 0)
    pltpu.sync_copy(out_v, out_hbm.at[pl.ds(base, _SEG_PER_TILE)])


def _sc_gather(p2, p3, p4, s2d2, s2d3, s2d4, sub2, sub3, sub4):
    mesh = plsc.VectorSubcoreMesh(
        core_axis_name="c", subcore_axis_name="s",
        num_cores=_SC_CORES, num_subcores=_SC_SUBCORES)
    fn = pl.kernel(
        _sc_body,
        out_type=jax.ShapeDtypeStruct((_NB, _CONFS), jnp.float32),
        mesh=mesh,
        compiler_params=pltpu.CompilerParams(needs_layout_passes=False),
        scratch_types=[
            pltpu.VMEM((1024,), jnp.int32),
            pltpu.VMEM((2048,), jnp.int32),
            pltpu.VMEM((3072,), jnp.int32),
            pltpu.VMEM((_NQ,), jnp.int32),
            pltpu.VMEM((_NQ, 128), jnp.int32),
            pltpu.VMEM((_NQ,), jnp.int32),
            pltpu.VMEM((_NQ,), jnp.int32),
            pltpu.VMEM((_NQ,), jnp.int32),
            pltpu.VMEM((_NQ,), jnp.int32),
            pltpu.VMEM((_NQ,), jnp.int32),
            pltpu.VMEM((_NQ,), jnp.int32),
            pltpu.VMEM((_NQ, 128), jnp.float32),
            pltpu.VMEM((_NQ, 128), jnp.float32),
            pltpu.VMEM((_NQ, 128), jnp.float32),
            pltpu.VMEM((_SEG_PER_TILE, _CONFS), jnp.float32),
            pltpu.SemaphoreType.DMA,
            pltpu.SemaphoreType.DMA,
        ],
    )
    return fn(p2, p3, p4, s2d2, s2d3, s2d4, sub2, sub3, sub4)


def kernel(k2, eq2, x2, seg2, k3, eq3, x3, seg3, k4, x4, seg4):
    n2, n3, n4 = x2.shape[0], x3.shape[0], x4.shape[0]
    p2, p3, p4 = _prefix(
        k2.reshape(n2, 1), eq2.reshape(n2, 1), x2,
        k3.reshape(n3, 1), eq3.reshape(n3, 1), x3,
        k4, x4)
    seg2 = seg2.astype(jnp.int32)
    seg3 = seg3.astype(jnp.int32)
    seg4 = seg4.astype(jnp.int32)
    return _sc_gather(
        p2, p3, p4,
        seg2.reshape(n2 // 128, 128),
        seg3.reshape(n3 // 128, 128),
        seg4.reshape(n4 // 128, 128),
        seg2[::128], seg3[::128], seg4[::128])


# speedup vs baseline: 2.0980x; 1.2196x over previous
import jax
import jax.numpy as jnp
from jax import lax
from jax.experimental import pallas as pl
from jax.experimental.pallas import tpu as pltpu
from jax.experimental.pallas import tpu_sc as plsc

_NB = 4096
_CONFS = 32
_R2, _R3, _R4 = 1024, 2048, 3072

_SC_CORES = 2
_SC_SUBCORES = 16
_SC_TILES = _SC_CORES * _SC_SUBCORES
_SEG_PER_TILE = _NB // _SC_TILES
_NQ = 144
_NCHUNK = _NQ // 16


def _cumsum_rows(x):
    rows = x.shape[0]
    k = 1
    while k < rows:
        shifted = jnp.concatenate(
            [jnp.zeros((k, x.shape[1]), x.dtype), x[: rows - k]], axis=0)
        x = x + shifted
        k *= 2
    return x


def _shift_lanes(x, k):
    return jnp.concatenate(
        [jnp.zeros((x.shape[0], k), x.dtype), x[:, : x.shape[1] - k]], axis=1)


def _prefix_body(k2_ref, eq2_ref, x2_ref, k3_ref, eq3_ref, x3_ref,
                 k4_ref, x4_ref, p2_ref, p3_ref, p4_ref,
                 c2_ref, c3_ref, c4_ref):
    b = pl.program_id(0)

    @pl.when(b == 0)
    def _zero():
        c2_ref[...] = jnp.zeros_like(c2_ref)
        c3_ref[...] = jnp.zeros_like(c3_ref)
        c4_ref[...] = jnp.zeros_like(c4_ref)
        p2_ref[...] = jnp.zeros_like(p2_ref)
        p3_ref[...] = jnp.zeros_like(p3_ref)
        p4_ref[...] = jnp.zeros_like(p4_ref)

    @pl.when(b > 0)
    def _compute():
        def pack(x, rows):
            q = rows // 4
            return jnp.concatenate(
                [x[0:q], x[q:2 * q], x[2 * q:3 * q], x[3 * q:4 * q]], axis=1)

        def emit(p_ref, c_ref, y, q):
            y = _cumsum_rows(y)
            last = y[q - 1:q, :]
            a = _shift_lanes(last, 32)
            off = a + _shift_lanes(a, 32) + _shift_lanes(a, 64)
            p_ref[...] = y + off + c_ref[...]
            tot = (off + last)[:, 96:128]
            c_ref[...] = c_ref[...] + jnp.concatenate(
                [tot, tot, tot, tot], axis=1)

        d2 = pack(x2_ref[...], _R2) - eq2_ref[...]
        emit(p2_ref, c2_ref, 0.5 * k2_ref[...] * (d2 * d2), _R2 // 4)

        d3 = pack(x3_ref[...], _R3) - eq3_ref[...]
        emit(p3_ref, c3_ref, 0.5 * k3_ref[...] * (d3 * d3), _R3 // 4)

        c1 = jnp.cos(pack(x4_ref[...], _R4))
        t2 = 2.0 * c1 * c1 - 1.0
        t3 = 2.0 * c1 * t2 - c1
        t4 = 2.0 * c1 * t3 - t2
        t5 = 2.0 * c1 * t4 - t3
        t6 = 2.0 * c1 * t5 - t4
        k4 = k4_ref[...]
        q4 = _R4 // 4

        def kcol(j):
            return jnp.concatenate(
                [jnp.broadcast_to(k4[:, 6 * k + j:6 * k + j + 1],
                                  (q4, _CONFS)) for k in range(4)], axis=1)

        e4 = (kcol(0) * c1 + kcol(1) * t2 + kcol(2) * t3
              + kcol(3) * t4 + kcol(4) * t5 + kcol(5) * t6)
        emit(p4_ref, c4_ref, e4, _R4 // 4)


def _prefix(k2, eq2, x2, k3, eq3, x3, k4, x4):
    n2, n3, n4 = x2.shape[0], x3.shape[0], x4.shape[0]
    steps = n2 // _R2 + 1

    def shifted(r):
        return lambda b: (jnp.maximum(b - 1, 0), 0)

    return pl.pallas_call(
        _prefix_body,
        grid=(steps,),
        in_specs=[
            pl.BlockSpec((_R2 // 4, 128), shifted(_R2)),
            pl.BlockSpec((_R2 // 4, 128), shifted(_R2)),
            pl.BlockSpec((_R2, _CONFS), shifted(_R2)),
            pl.BlockSpec((_R3 // 4, 128), shifted(_R3)),
            pl.BlockSpec((_R3 // 4, 128), shifted(_R3)),
            pl.BlockSpec((_R3, _CONFS), shifted(_R3)),
            pl.BlockSpec((_R4 // 4, 24), shifted(_R4)),
            pl.BlockSpec((_R4, _CONFS), shifted(_R4)),
        ],
        out_specs=[
            pl.BlockSpec((_R2 // 4, 128), lambda b: (b, 0)),
            pl.BlockSpec((_R3 // 4, 128), lambda b: (b, 0)),
            pl.BlockSpec((_R4 // 4, 128), lambda b: (b, 0)),
        ],
        out_shape=[
            jax.ShapeDtypeStruct(((n2 + _R2) // 4, 128), jnp.float32),
            jax.ShapeDtypeStruct(((n3 + _R3) // 4, 128), jnp.float32),
            jax.ShapeDtypeStruct(((n4 + _R4) // 4, 128), jnp.float32),
        ],
        scratch_shapes=[
            pltpu.VMEM((1, 128), jnp.float32),
            pltpu.VMEM((1, 128), jnp.float32),
            pltpu.VMEM((1, 128), jnp.float32),
        ],
    )(k2, eq2, x2, k3, eq3, x3, k4, x4)


def _map2(jj):
    return (jj >> 10) * 256 + (jj & 255), ((jj & 1023) >> 8) * 32


def _map3(jj):
    return (jj >> 11) * 512 + (jj & 511), ((jj & 2047) >> 9) * 32


def _map4(jj):
    b = ((jj >> 10) * 21846) >> 16
    ul = jj - b * 3072
    k = ((ul >= 768).astype(jnp.int32) + (ul >= 1536).astype(jnp.int32)
         + (ul >= 2304).astype(jnp.int32))
    return b * 768 + (ul - k * 768), k * 32


def _sc_body(p2_hbm, p3_hbm, p4_hbm, s2d2_hbm, s2d3_hbm, s2d4_hbm,
             sub2_hbm, sub3_hbm, sub4_hbm, out_hbm,
             sub2_v, sub3_v, sub4_v, probe_v, segrows_v,
             km2_v, km3_v, km4_v, qr2_v, qr3_v, qr4_v,
             g2_v, g3_v, g4_v, out_v, sem_s, sem_p):
    wid = lax.axis_index("s") * _SC_CORES + lax.axis_index("c")
    base = wid * _SEG_PER_TILE

    pltpu.sync_copy(sub2_hbm, sub2_v)
    pltpu.sync_copy(sub3_hbm, sub3_v)
    pltpu.sync_copy(sub4_hbm, sub4_v)

    p_copies = []
    for sub_v, nsub, s2d_hbm, km_v, qr_v, p_hbm, g_v, radd, pmap in (
            (sub2_v, 1024, s2d2_hbm, km2_v, qr2_v, p2_hbm, g2_v, _R2 - 1, _map2),
            (sub3_v, 2048, s2d3_hbm, km3_v, qr3_v, p3_hbm, g3_v, _R3 - 1, _map3),
            (sub4_v, 3072, s2d4_hbm, km4_v, qr4_v, p4_hbm, g4_v, _R4 - 1, _map4)):

        def coarse(c, carry, sub_v=sub_v, nsub=nsub):
            s = base + c * 16 + lax.iota(jnp.int32, 16)
            lo = jnp.zeros((16,), jnp.int32)
            hi = jnp.full((16,), nsub, jnp.int32)
            for _ in range(12):
                mid = jnp.minimum((lo + hi) >> 1, nsub - 1)
                v = plsc.load_gather(sub_v, [mid])
                below = v < s
                lo = jnp.where(below, mid + 1, lo)
                hi = jnp.where(below, hi, mid)
            probe_v[pl.ds(c * 16, 16)] = jnp.maximum(lo - 1, 0)
            return carry

        lax.fori_loop(0, _NCHUNK, coarse, 0)

        pltpu.async_copy(s2d_hbm.at[probe_v.at[pl.ds(0, 128)]],
                         segrows_v.at[pl.ds(0, 128)], sem_s).wait()
        pltpu.async_copy(s2d_hbm.at[probe_v.at[pl.ds(128, 16)]],
                         segrows_v.at[pl.ds(128, 16)], sem_s).wait()

        def fine(c, carry, km_v=km_v, qr_v=qr_v, radd=radd, pmap=pmap):
            s = base + c * 16 + lax.iota(jnp.int32, 16)
            row = c * 16 + lax.iota(jnp.int32, 16)
            qb = probe_v[pl.ds(c * 16, 16)]
            lo = jnp.zeros((16,), jnp.int32)
            hi = jnp.full((16,), 128, jnp.int32)
            for _ in range(8):
                mid = jnp.minimum((lo + hi) >> 1, 127)
                v = plsc.load_gather(segrows_v, [row, mid])
                below = v < s
                lo = jnp.where(below, mid + 1, lo)
                hi = jnp.where(below, hi, mid)
            jj = qb * 128 + lo + radd
            prow, km = pmap(jj)
            qr_v[pl.ds(c * 16, 16)] = prow
            km_v[pl.ds(c * 16, 16)] = km
            return carry

        lax.fori_loop(0, _NCHUNK, fine, 0)

        p_copies.append(pltpu.async_copy(
            p_hbm.at[qr_v.at[pl.ds(0, 128)]], g_v.at[pl.ds(0, 128)], sem_p))
        p_copies.append(pltpu.async_copy(
            p_hbm.at[qr_v.at[pl.ds(128, 16)]], g_v.at[pl.ds(128, 16)], sem_p))

    for cp in p_copies:
        cp.wait()

    def diff(c, carry):
        rlo = c * 16 + lax.iota(jnp.int32, 16)
        rhi = rlo + 1
        jmlo2 = km2_v[pl.ds(c * 16, 16)]
        jmhi2 = km2_v[pl.ds(c * 16 + 1, 16)]
        jmlo3 = km3_v[pl.ds(c * 16, 16)]
        jmhi3 = km3_v[pl.ds(c * 16 + 1, 16)]
        jmlo4 = km4_v[pl.ds(c * 16, 16)]
        jmhi4 = km4_v[pl.ds(c * 16 + 1, 16)]
        for cc in range(_CONFS):
            col = jnp.full((16,), cc, jnp.int32)
            acc = (plsc.load_gather(g2_v, [rhi, jmhi2 + cc])
                   - plsc.load_gather(g2_v, [rlo, jmlo2 + cc])
                   + plsc.load_gather(g3_v, [rhi, jmhi3 + cc])
                   - plsc.load_gather(g3_v, [rlo, jmlo3 + cc])
                   + plsc.load_gather(g4_v, [rhi, jmhi4 + cc])
                   - plsc.load_gather(g4_v, [rlo, jmlo4 + cc]))
            plsc.store_scatter(out_v, [rlo, col], acc)
        return carry

    lax.fori_loop(0, _SEG_PER_TILE // 16, diff, 0)
    pltpu.sync_copy(out_v, out_hbm.at[pl.ds(base, _SEG_PER_TILE)])


def _sc_gather(p2, p3, p4, s2d2, s2d3, s2d4, sub2, sub3, sub4):
    mesh = plsc.VectorSubcoreMesh(
        core_axis_name="c", subcore_axis_name="s",
        num_cores=_SC_CORES, num_subcores=_SC_SUBCORES)
    fn = pl.kernel(
        _sc_body,
        out_type=jax.ShapeDtypeStruct((_NB, _CONFS), jnp.float32),
        mesh=mesh,
        compiler_params=pltpu.CompilerParams(needs_layout_passes=False),
        scratch_types=[
            pltpu.VMEM((1024,), jnp.int32),
            pltpu.VMEM((2048,), jnp.int32),
            pltpu.VMEM((3072,), jnp.int32),
            pltpu.VMEM((_NQ,), jnp.int32),
            pltpu.VMEM((_NQ, 128), jnp.int32),
            pltpu.VMEM((_NQ,), jnp.int32),
            pltpu.VMEM((_NQ,), jnp.int32),
            pltpu.VMEM((_NQ,), jnp.int32),
            pltpu.VMEM((_NQ,), jnp.int32),
            pltpu.VMEM((_NQ,), jnp.int32),
            pltpu.VMEM((_NQ,), jnp.int32),
            pltpu.VMEM((_NQ, 128), jnp.float32),
            pltpu.VMEM((_NQ, 128), jnp.float32),
            pltpu.VMEM((_NQ, 128), jnp.float32),
            pltpu.VMEM((_SEG_PER_TILE, _CONFS), jnp.float32),
            pltpu.SemaphoreType.DMA,
            pltpu.SemaphoreType.DMA,
        ],
    )
    return fn(p2, p3, p4, s2d2, s2d3, s2d4, sub2, sub3, sub4)


def _coeff_packed(k, n, r):
    q = r // 4
    t = k.reshape(n // r, 4, q).transpose(0, 2, 1).reshape(n // 4, 4, 1)
    return jnp.broadcast_to(t, (n // 4, 4, _CONFS)).reshape(n // 4, 128)


def kernel(k2, eq2, x2, seg2, k3, eq3, x3, seg3, k4, x4, seg4):
    n2, n3, n4 = x2.shape[0], x3.shape[0], x4.shape[0]
    q4 = _R4 // 4
    k4r = (k4.reshape(n4 // _R4, 4, q4, 6).transpose(0, 2, 1, 3)
           .reshape(n4 // 4, 24))
    p2, p3, p4 = _prefix(
        _coeff_packed(k2, n2, _R2), _coeff_packed(eq2, n2, _R2), x2,
        _coeff_packed(k3, n3, _R3), _coeff_packed(eq3, n3, _R3), x3,
        k4r, x4)
    seg2 = seg2.astype(jnp.int32)
    seg3 = seg3.astype(jnp.int32)
    seg4 = seg4.astype(jnp.int32)
    return _sc_gather(
        p2, p3, p4,
        seg2.reshape(n2 // 128, 128),
        seg3.reshape(n3 // 128, 128),
        seg4.reshape(n4 // 128, 128),
        seg2[::128], seg3[::128], seg4[::128])


# speedup vs baseline: 2.1167x; 1.0089x over previous
import jax
import jax.numpy as jnp
from jax import lax
from jax.experimental import pallas as pl
from jax.experimental.pallas import tpu as pltpu
from jax.experimental.pallas import tpu_sc as plsc

_NB = 4096
_CONFS = 32
_R2, _R3, _R4 = 2048, 4096, 6144

_SC_CORES = 2
_SC_SUBCORES = 16
_SC_TILES = _SC_CORES * _SC_SUBCORES
_SEG_PER_TILE = _NB // _SC_TILES
_NQ = 144
_NCHUNK = _NQ // 16


def _cumsum_rows(x):
    rows = x.shape[0]
    k = 1
    while k < rows:
        shifted = jnp.concatenate(
            [jnp.zeros((k, x.shape[1]), x.dtype), x[: rows - k]], axis=0)
        x = x + shifted
        k *= 2
    return x


def _shift_lanes(x, k):
    return jnp.concatenate(
        [jnp.zeros((x.shape[0], k), x.dtype), x[:, : x.shape[1] - k]], axis=1)


def _prefix_body(kc2_ref, x2_ref, kc3_ref, x3_ref,
                 k4_ref, x4_ref, p2_ref, p3_ref, p4_ref,
                 c2_ref, c3_ref, c4_ref):
    b = pl.program_id(0)

    @pl.when(b == 0)
    def _zero():
        c2_ref[...] = jnp.zeros_like(c2_ref)
        c3_ref[...] = jnp.zeros_like(c3_ref)
        c4_ref[...] = jnp.zeros_like(c4_ref)
        p2_ref[...] = jnp.zeros_like(p2_ref)
        p3_ref[...] = jnp.zeros_like(p3_ref)
        p4_ref[...] = jnp.zeros_like(p4_ref)

    @pl.when(b > 0)
    def _compute():
        def pack(x, rows):
            q = rows // 4
            return jnp.concatenate(
                [x[0:q], x[q:2 * q], x[2 * q:3 * q], x[3 * q:4 * q]], axis=1)

        def emit(p_ref, c_ref, y, q):
            y = _cumsum_rows(y)
            last = y[q - 1:q, :]
            a = _shift_lanes(last, 32)
            off = a + _shift_lanes(a, 32) + _shift_lanes(a, 64)
            p_ref[...] = y + off + c_ref[...]
            tot = (off + last)[:, 96:128]
            c_ref[...] = c_ref[...] + jnp.concatenate(
                [tot, tot, tot, tot], axis=1)

        def bcast4(blk, idxs, q):
            return jnp.concatenate(
                [jnp.broadcast_to(blk[:, i:i + 1], (q, _CONFS))
                 for i in idxs], axis=1)

        kc2 = kc2_ref[...]
        d2 = pack(x2_ref[...], _R2) - bcast4(kc2, [4, 5, 6, 7], _R2 // 4)
        emit(p2_ref, c2_ref,
             0.5 * bcast4(kc2, [0, 1, 2, 3], _R2 // 4) * (d2 * d2), _R2 // 4)

        kc3 = kc3_ref[...]
        d3 = pack(x3_ref[...], _R3) - bcast4(kc3, [4, 5, 6, 7], _R3 // 4)
        emit(p3_ref, c3_ref,
             0.5 * bcast4(kc3, [0, 1, 2, 3], _R3 // 4) * (d3 * d3), _R3 // 4)

        c1 = jnp.cos(pack(x4_ref[...], _R4))
        t2 = 2.0 * c1 * c1 - 1.0
        t3 = 2.0 * c1 * t2 - c1
        t4 = 2.0 * c1 * t3 - t2
        t5 = 2.0 * c1 * t4 - t3
        t6 = 2.0 * c1 * t5 - t4
        k4 = k4_ref[...]
        q4 = _R4 // 4

        def kcol(j):
            return bcast4(k4, [6 * k + j for k in range(4)], q4)

        e4 = (kcol(0) * c1 + kcol(1) * t2 + kcol(2) * t3
              + kcol(3) * t4 + kcol(4) * t5 + kcol(5) * t6)
        emit(p4_ref, c4_ref, e4, _R4 // 4)


def _prefix(kc2, x2, kc3, x3, k4r, x4):
    n2, n3, n4 = x2.shape[0], x3.shape[0], x4.shape[0]
    steps = n2 // _R2 + 1

    def shifted(r):
        return lambda b: (jnp.maximum(b - 1, 0), 0)

    return pl.pallas_call(
        _prefix_body,
        grid=(steps,),
        in_specs=[
            pl.BlockSpec((_R2 // 4, 8), shifted(_R2)),
            pl.BlockSpec((_R2, _CONFS), shifted(_R2)),
            pl.BlockSpec((_R3 // 4, 8), shifted(_R3)),
            pl.BlockSpec((_R3, _CONFS), shifted(_R3)),
            pl.BlockSpec((_R4 // 4, 24), shifted(_R4)),
            pl.BlockSpec((_R4, _CONFS), shifted(_R4)),
        ],
        out_specs=[
            pl.BlockSpec((_R2 // 4, 128), lambda b: (b, 0)),
            pl.BlockSpec((_R3 // 4, 128), lambda b: (b, 0)),
            pl.BlockSpec((_R4 // 4, 128), lambda b: (b, 0)),
        ],
        out_shape=[
            jax.ShapeDtypeStruct(((n2 + _R2) // 4, 128), jnp.float32),
            jax.ShapeDtypeStruct(((n3 + _R3) // 4, 128), jnp.float32),
            jax.ShapeDtypeStruct(((n4 + _R4) // 4, 128), jnp.float32),
        ],
        scratch_shapes=[
            pltpu.VMEM((1, 128), jnp.float32),
            pltpu.VMEM((1, 128), jnp.float32),
            pltpu.VMEM((1, 128), jnp.float32),
        ],
    )(kc2, x2, kc3, x3, k4r, x4)


def _map2(jj):
    return (jj >> 11) * 512 + (jj & 511), ((jj & 2047) >> 9) * 32


def _map3(jj):
    return (jj >> 12) * 1024 + (jj & 1023), ((jj & 4095) >> 10) * 32


def _map4(jj):
    b = ((jj >> 11) * 21846) >> 16
    ul = jj - b * 6144
    k = ((ul >= 1536).astype(jnp.int32) + (ul >= 3072).astype(jnp.int32)
         + (ul >= 4608).astype(jnp.int32))
    return b * 1536 + (ul - k * 1536), k * 32


def _sc_body(p2_hbm, p3_hbm, p4_hbm, s2d2_hbm, s2d3_hbm, s2d4_hbm,
             sub2_hbm, sub3_hbm, sub4_hbm, out_hbm,
             sub2_v, sub3_v, sub4_v, probe_v, segrows_v,
             km2_v, km3_v, km4_v, qr2_v, qr3_v, qr4_v,
             g2_v, g3_v, g4_v, out_v, sem_s, sem_p):
    wid = lax.axis_index("s") * _SC_CORES + lax.axis_index("c")
    base = wid * _SEG_PER_TILE

    pltpu.sync_copy(sub2_hbm, sub2_v)
    pltpu.sync_copy(sub3_hbm, sub3_v)
    pltpu.sync_copy(sub4_hbm, sub4_v)

    p_copies = []
    for sub_v, nsub, s2d_hbm, km_v, qr_v, p_hbm, g_v, radd, pmap in (
            (sub2_v, 1024, s2d2_hbm, km2_v, qr2_v, p2_hbm, g2_v, _R2 - 1, _map2),
            (sub3_v, 2048, s2d3_hbm, km3_v, qr3_v, p3_hbm, g3_v, _R3 - 1, _map3),
            (sub4_v, 3072, s2d4_hbm, km4_v, qr4_v, p4_hbm, g4_v, _R4 - 1, _map4)):

        def coarse(c, carry, sub_v=sub_v, nsub=nsub):
            s = base + c * 16 + lax.iota(jnp.int32, 16)
            lo = jnp.zeros((16,), jnp.int32)
            hi = jnp.full((16,), nsub, jnp.int32)
            for _ in range(12):
                mid = jnp.minimum((lo + hi) >> 1, nsub - 1)
                v = plsc.load_gather(sub_v, [mid])
                below = v < s
                lo = jnp.where(below, mid + 1, lo)
                hi = jnp.where(below, hi, mid)
            probe_v[pl.ds(c * 16, 16)] = jnp.maximum(lo - 1, 0)
            return carry

        lax.fori_loop(0, _NCHUNK, coarse, 0)

        pltpu.async_copy(s2d_hbm.at[probe_v.at[pl.ds(0, 128)]],
                         segrows_v.at[pl.ds(0, 128)], sem_s).wait()
        pltpu.async_copy(s2d_hbm.at[probe_v.at[pl.ds(128, 16)]],
                         segrows_v.at[pl.ds(128, 16)], sem_s).wait()

        def fine(c, carry, km_v=km_v, qr_v=qr_v, radd=radd, pmap=pmap):
            s = base + c * 16 + lax.iota(jnp.int32, 16)
            row = c * 16 + lax.iota(jnp.int32, 16)
            qb = probe_v[pl.ds(c * 16, 16)]
            lo = jnp.zeros((16,), jnp.int32)
            hi = jnp.full((16,), 128, jnp.int32)
            for _ in range(8):
                mid = jnp.minimum((lo + hi) >> 1, 127)
                v = plsc.load_gather(segrows_v, [row, mid])
                below = v < s
                lo = jnp.where(below, mid + 1, lo)
                hi = jnp.where(below, hi, mid)
            jj = qb * 128 + lo + radd
            prow, km = pmap(jj)
            qr_v[pl.ds(c * 16, 16)] = prow
            km_v[pl.ds(c * 16, 16)] = km
            return carry

        lax.fori_loop(0, _NCHUNK, fine, 0)

        p_copies.append(pltpu.async_copy(
            p_hbm.at[qr_v.at[pl.ds(0, 128)]], g_v.at[pl.ds(0, 128)], sem_p))
        p_copies.append(pltpu.async_copy(
            p_hbm.at[qr_v.at[pl.ds(128, 16)]], g_v.at[pl.ds(128, 16)], sem_p))

    for cp in p_copies:
        cp.wait()

    def diff(c, carry):
        rlo = c * 16 + lax.iota(jnp.int32, 16)
        rhi = rlo + 1
        jmlo2 = km2_v[pl.ds(c * 16, 16)]
        jmhi2 = km2_v[pl.ds(c * 16 + 1, 16)]
        jmlo3 = km3_v[pl.ds(c * 16, 16)]
        jmhi3 = km3_v[pl.ds(c * 16 + 1, 16)]
        jmlo4 = km4_v[pl.ds(c * 16, 16)]
        jmhi4 = km4_v[pl.ds(c * 16 + 1, 16)]
        for cc in range(_CONFS):
            col = jnp.full((16,), cc, jnp.int32)
            acc = (plsc.load_gather(g2_v, [rhi, jmhi2 + cc])
                   - plsc.load_gather(g2_v, [rlo, jmlo2 + cc])
                   + plsc.load_gather(g3_v, [rhi, jmhi3 + cc])
                   - plsc.load_gather(g3_v, [rlo, jmlo3 + cc])
                   + plsc.load_gather(g4_v, [rhi, jmhi4 + cc])
                   - plsc.load_gather(g4_v, [rlo, jmlo4 + cc]))
            plsc.store_scatter(out_v, [rlo, col], acc)
        return carry

    lax.fori_loop(0, _SEG_PER_TILE // 16, diff, 0)
    pltpu.sync_copy(out_v, out_hbm.at[pl.ds(base, _SEG_PER_TILE)])


def _sc_gather(p2, p3, p4, s2d2, s2d3, s2d4, sub2, sub3, sub4):
    mesh = plsc.VectorSubcoreMesh(
        core_axis_name="c", subcore_axis_name="s",
        num_cores=_SC_CORES, num_subcores=_SC_SUBCORES)
    fn = pl.kernel(
        _sc_body,
        out_type=jax.ShapeDtypeStruct((_NB, _CONFS), jnp.float32),
        mesh=mesh,
        compiler_params=pltpu.CompilerParams(needs_layout_passes=False),
        scratch_types=[
            pltpu.VMEM((1024,), jnp.int32),
            pltpu.VMEM((2048,), jnp.int32),
            pltpu.VMEM((3072,), jnp.int32),
            pltpu.VMEM((_NQ,), jnp.int32),
            pltpu.VMEM((_NQ, 128), jnp.int32),
            pltpu.VMEM((_NQ,), jnp.int32),
            pltpu.VMEM((_NQ,), jnp.int32),
            pltpu.VMEM((_NQ,), jnp.int32),
            pltpu.VMEM((_NQ,), jnp.int32),
            pltpu.VMEM((_NQ,), jnp.int32),
            pltpu.VMEM((_NQ,), jnp.int32),
            pltpu.VMEM((_NQ, 128), jnp.float32),
            pltpu.VMEM((_NQ, 128), jnp.float32),
            pltpu.VMEM((_NQ, 128), jnp.float32),
            pltpu.VMEM((_SEG_PER_TILE, _CONFS), jnp.float32),
            pltpu.SemaphoreType.DMA,
            pltpu.SemaphoreType.DMA,
        ],
    )
    return fn(p2, p3, p4, s2d2, s2d3, s2d4, sub2, sub3, sub4)


def _coeff_cols(k, n, r):
    q = r // 4
    return k.reshape(n // r, 4, q).transpose(0, 2, 1).reshape(n // 4, 4)


def kernel(k2, eq2, x2, seg2, k3, eq3, x3, seg3, k4, x4, seg4):
    n2, n3, n4 = x2.shape[0], x3.shape[0], x4.shape[0]
    q4 = _R4 // 4
    k4r = (k4.reshape(n4 // _R4, 4, q4, 6).transpose(0, 2, 1, 3)
           .reshape(n4 // 4, 24))
    kc2 = jnp.concatenate(
        [_coeff_cols(k2, n2, _R2), _coeff_cols(eq2, n2, _R2)], axis=1)
    kc3 = jnp.concatenate(
        [_coeff_cols(k3, n3, _R3), _coeff_cols(eq3, n3, _R3)], axis=1)
    p2, p3, p4 = _prefix(kc2, x2, kc3, x3, k4r, x4)
    seg2 = seg2.astype(jnp.int32)
    seg3 = seg3.astype(jnp.int32)
    seg4 = seg4.astype(jnp.int32)
    return _sc_gather(
        p2, p3, p4,
        seg2.reshape(n2 // 128, 128),
        seg3.reshape(n3 // 128, 128),
        seg4.reshape(n4 // 128, 128),
        seg2[::128], seg3[::128], seg4[::128])
